# baseline (speedup 1.0000x reference)
import functools
import jax
import jax.numpy as jnp
from jax import lax
from jax.experimental import pallas as pl
from jax.experimental.pallas import tpu as pltpu
from jax.experimental.pallas import tpu_sc as plsc

D = 64
DP = 128
SCALE = 8.0
HALVES = ((0, 104), (104, 96))
NC, NS = 2, 16
NW = NC * NS
NBUF = 3


def kernel(token, embeddings):
    BATCH, SEQ = token.shape
    rows_per_w = BATCH // NW
    tok2d = token.astype(jnp.int32)
    tblp = jnp.pad(embeddings, ((0, 0), (0, DP - D)))

    mesh = plsc.VectorSubcoreMesh(
        core_axis_name="c", subcore_axis_name="s",
        num_cores=NC, num_subcores=NS)

    @functools.partial(
        pl.kernel,
        out_type=jax.ShapeDtypeStruct((BATCH, SEQ, DP), jnp.float32),
        mesh=mesh,
        compiler_params=pltpu.CompilerParams(
            use_tc_tiling_on_sc=False, skip_device_barrier=True),
        scratch_types=[
            pltpu.VMEM((rows_per_w, SEQ), jnp.int32),
            pltpu.VMEM((NBUF, SEQ, DP), jnp.float32),
            pltpu.SemaphoreType.DMA((NBUF,)),
            pltpu.SemaphoreType.DMA((NBUF,)),
        ],
    )
    def emb(tok_hbm, table_hbm, out_hbm, idx_v, raw_v, gsem, ssem):
        wid = lax.axis_index("s") * NC + lax.axis_index("c")
        brow = wid * rows_per_w

        pltpu.sync_copy(tok_hbm.at[pl.ds(brow, rows_per_w)], idx_v)

        def start_gathers(r, s):
            for off, ln in HALVES:
                pltpu.async_copy(
                    table_hbm.at[idx_v.at[r, pl.ds(off, ln)]],
                    raw_v.at[s, pl.ds(off, ln)],
                    gsem.at[s])

        def wait_gathers(s):
            for off, ln in HALVES:
                pltpu.make_async_copy(
                    table_hbm.at[pl.ds(0, ln)],
                    raw_v.at[s, pl.ds(off, ln)],
                    gsem.at[s]).wait()

        def start_store(r, s):
            pltpu.async_copy(raw_v.at[s], out_hbm.at[brow + r], ssem.at[s])

        def wait_store(s):
            pltpu.make_async_copy(raw_v.at[s], out_hbm.at[brow],
                                  ssem.at[s]).wait()

        def scale(s):
            def row_body(i, c):
                for k in range(D // 16):
                    sl = pl.ds(k * 16, 16)
                    raw_v[s, i, sl] = raw_v[s, i, sl] * SCALE
                return c
            lax.fori_loop(0, SEQ, row_body, 0)

        def step(r, s, do_issue):
            wait_gathers(s)
            scale(s)
            start_store(r, s)
            if do_issue:
                wait_store(s)
                start_gathers(r + NBUF, s)

        for s in range(NBUF):
            start_gathers(s, s)
        for r in range(NBUF):
            step(r, r % NBUF, True)

        n_main = (rows_per_w - 2 * NBUF) // NBUF
        def main_body(g, carry):
            for b in range(NBUF):
                step(g * NBUF + b, b, True)
            return carry
        lax.fori_loop(1, 1 + n_main, main_body, 0)

        done = NBUF + n_main * NBUF
        for r in range(done, rows_per_w):
            step(r, r % NBUF, r + NBUF < rows_per_w)

        for s in range(NBUF):
            wait_store(s)

    out3 = emb(tok2d, tblp)
    return out3[:, :, :D]
